# Initial kernel scaffold; baseline (speedup 1.0000x reference)
import jax
import jax.numpy as jnp
from jax.experimental import pallas as pl


def kernel(indices, table):
    raise NotImplementedError("write your pallas kernel here")


# speedup vs baseline: 1.1132x; 1.1132x over previous
import functools

import jax
import jax.numpy as jnp
from jax import lax
from jax.experimental import pallas as pl
from jax.experimental.pallas import tpu as pltpu
from jax.experimental.pallas import tpu_sc as plsc

_NC = 2
_NS = 16
_NW = _NC * _NS
_W = 128
_CHUNK = 512
_KW = _CHUNK // _W


def kernel(indices, table):
    b, s = indices.shape
    n = b * s
    d = table.shape[1]
    per = n // _NW
    nwin = per // _W
    nchunks = per // _CHUNK
    idx2d = indices.reshape(n // _W, _W)
    mesh = plsc.VectorSubcoreMesh(core_axis_name="c", subcore_axis_name="s")

    @functools.partial(
        pl.kernel,
        mesh=mesh,
        compiler_params=pltpu.CompilerParams(use_tc_tiling_on_sc=False),
        out_type=jax.ShapeDtypeStruct((n, d), table.dtype),
        scratch_types=[
            pltpu.VMEM((nwin, _W), jnp.int32),
            pltpu.VMEM((2, _CHUNK, d), jnp.float32),
            pltpu.SemaphoreType.DMA,
            pltpu.SemaphoreType.DMA,
        ],
    )
    def _gather(table_hbm, idx_hbm, out_hbm, idx_v, rows_v, gsem, osem):
        wid = lax.axis_index("c") * _NS + lax.axis_index("s")
        wbase = wid * nwin
        rbase = wid * per
        pltpu.sync_copy(idx_hbm.at[pl.ds(wbase, nwin)], idx_v)

        def issue_gathers(c, buf):
            for j in range(_KW):
                pltpu.async_copy(
                    table_hbm.at[idx_v.at[c * _KW + j]],
                    rows_v.at[buf].at[pl.ds(j * _W, _W)],
                    gsem,
                )

        def drain_gathers(buf):
            for j in range(_KW):
                pltpu.make_async_copy(
                    table_hbm.at[idx_v.at[j]],
                    rows_v.at[buf].at[pl.ds(j * _W, _W)],
                    gsem,
                ).wait()

        def start_out(c, buf):
            pltpu.async_copy(
                rows_v.at[buf], out_hbm.at[pl.ds(rbase + c * _CHUNK, _CHUNK)], osem
            )

        def wait_out(buf):
            pltpu.make_async_copy(
                rows_v.at[buf], out_hbm.at[pl.ds(rbase, _CHUNK)], osem
            ).wait()

        issue_gathers(0, 0)
        issue_gathers(1, 1)

        @pl.loop(0, nchunks - 2, step=2)
        def _(ci):
            drain_gathers(0)
            start_out(ci, 0)
            drain_gathers(1)
            start_out(ci + 1, 1)
            wait_out(0)
            issue_gathers(ci + 2, 0)
            wait_out(1)
            issue_gathers(ci + 3, 1)

        drain_gathers(0)
        start_out(nchunks - 2, 0)
        drain_gathers(1)
        start_out(nchunks - 1, 1)
        wait_out(0)
        wait_out(1)

    out = _gather(table, idx2d)
    return out.reshape(b, s, d)


# speedup vs baseline: 1.5343x; 1.3783x over previous
import functools

import jax
import jax.numpy as jnp
from jax import lax
from jax.experimental import pallas as pl
from jax.experimental.pallas import tpu as pltpu
from jax.experimental.pallas import tpu_sc as plsc

_NC = 2
_NS = 16
_NW = _NC * _NS


def kernel(indices, table):
    b, s = indices.shape
    d = table.shape[1]
    nd = d // 8
    nl = b // 128
    lpw = nl // _NW
    nunits = s * lpw
    idx_t = indices.T
    mesh = plsc.VectorSubcoreMesh(core_axis_name="c", subcore_axis_name="s")

    @functools.partial(
        pl.kernel,
        mesh=mesh,
        compiler_params=pltpu.CompilerParams(
            use_tc_tiling_on_sc=False, needs_layout_passes=False
        ),
        out_type=jax.ShapeDtypeStruct((s, nd, nl, 8, 128), table.dtype),
        scratch_types=[
            pltpu.VMEM((s, lpw * 128), jnp.int32),
            pltpu.VMEM((2, 128, d), jnp.float32),
            pltpu.VMEM((2, nd, 8, 128), jnp.float32),
            pltpu.SemaphoreType.DMA,
            pltpu.SemaphoreType.DMA,
            pltpu.SemaphoreType.DMA,
            pltpu.SemaphoreType.DMA,
        ],
    )
    def _gather(table_hbm, idx_hbm, out_hbm, idx_v, rows_v, tiles_v,
                gsem0, gsem1, osem0, osem1):
        gsems = (gsem0, gsem1)
        osems = (osem0, osem1)
        wid = lax.axis_index("c") * _NS + lax.axis_index("s")
        lbase = wid * lpw
        pltpu.sync_copy(idx_hbm.at[:, pl.ds(lbase * 128, lpw * 128)], idx_v)

        def issue_gather(u, buf):
            slot, j = u // lpw, u % lpw
            pltpu.async_copy(
                table_hbm.at[idx_v.at[slot].at[pl.ds(j * 128, 128)]],
                rows_v.at[buf],
                gsems[buf],
            )

        def drain_gather(buf):
            pltpu.make_async_copy(
                table_hbm.at[idx_v.at[0].at[pl.ds(0, 128)]], rows_v.at[buf],
                gsems[buf],
            ).wait()

        iota16 = jnp.arange(16, dtype=jnp.int32)

        def transpose(buf):
            for si in range(nd):
                for r in range(8):
                    dd = jnp.full((16,), si * 8 + r, jnp.int32)
                    for k in range(8):
                        tiles_v.at[buf, si, r, pl.ds(k * 16, 16)][...] = (
                            plsc.load_gather(rows_v.at[buf], [iota16 + k * 16, dd])
                        )

        def write_out(u, buf):
            slot, j = u // lpw, u % lpw
            for si in range(nd):
                pltpu.async_copy(
                    tiles_v.at[buf, si], out_hbm.at[slot, si, lbase + j],
                    osems[buf],
                )

        def wait_out(buf):
            for si in range(nd):
                pltpu.make_async_copy(
                    tiles_v.at[buf, si], out_hbm.at[0, si, 0], osems[buf]
                ).wait()

        issue_gather(0, 0)
        issue_gather(1, 1)
        drain_gather(0)
        transpose(0)
        issue_gather(2, 0)
        write_out(0, 0)
        drain_gather(1)
        transpose(1)
        issue_gather(3, 1)
        write_out(1, 1)

        @pl.loop(2, nunits - 2, step=2)
        def _(u):
            drain_gather(0)
            wait_out(0)
            transpose(0)
            issue_gather(u + 2, 0)
            write_out(u, 0)
            drain_gather(1)
            wait_out(1)
            transpose(1)
            issue_gather(u + 3, 1)
            write_out(u + 1, 1)

        drain_gather(0)
        wait_out(0)
        transpose(0)
        write_out(nunits - 2, 0)
        drain_gather(1)
        wait_out(1)
        transpose(1)
        write_out(nunits - 1, 1)
        wait_out(0)
        wait_out(1)

    kout = _gather(table, idx_t)
    return kout.transpose(2, 4, 0, 1, 3).reshape(b, s, d)


# speedup vs baseline: 1.6423x; 1.0704x over previous
import functools

import jax
import jax.numpy as jnp
from jax import lax
from jax.experimental import pallas as pl
from jax.experimental.pallas import tpu as pltpu
from jax.experimental.pallas import tpu_sc as plsc

_NC = 2
_NS = 16
_NW = _NC * _NS
_G = 8


def kernel(indices, table):
    b, s = indices.shape
    d = table.shape[1]
    nd = d // 8
    nl = b // 128
    lpw = nl // _NW
    nunits = s * lpw
    idx_t = indices.T
    mesh = plsc.VectorSubcoreMesh(core_axis_name="c", subcore_axis_name="s")

    @functools.partial(
        pl.kernel,
        mesh=mesh,
        compiler_params=pltpu.CompilerParams(
            use_tc_tiling_on_sc=False, needs_layout_passes=False
        ),
        out_type=jax.ShapeDtypeStruct((s, nd, nl, 8, 128), table.dtype),
        scratch_types=[
            pltpu.VMEM((s, lpw * 128), jnp.int32),
            pltpu.VMEM((_G, 128, d), jnp.float32),
            pltpu.VMEM((2, nd, 8, 128), jnp.float32),
        ]
        + [pltpu.SemaphoreType.DMA] * (_G + 2),
    )
    def _gather(table_hbm, idx_hbm, out_hbm, idx_v, rows_v, tiles_v, *sems):
        gsems = sems[:_G]
        osems = sems[_G:]
        wid = lax.axis_index("c") * _NS + lax.axis_index("s")
        lbase = wid * lpw
        pltpu.sync_copy(idx_hbm.at[:, pl.ds(lbase * 128, lpw * 128)], idx_v)

        def issue_gather(u, g):
            slot, j = u // lpw, u % lpw
            pltpu.async_copy(
                table_hbm.at[idx_v.at[slot].at[pl.ds(j * 128, 128)]],
                rows_v.at[g],
                gsems[g],
            )

        def drain_gather(g):
            pltpu.make_async_copy(
                table_hbm.at[idx_v.at[0].at[pl.ds(0, 128)]], rows_v.at[g],
                gsems[g],
            ).wait()

        iota16 = jnp.arange(16, dtype=jnp.int32)

        def transpose(g, tb):
            @pl.loop(0, d)
            def _(dd):
                si, r = dd // 8, dd % 8
                ddv = jnp.full((16,), 0, jnp.int32) + dd
                for k in range(8):
                    tiles_v.at[tb, si, r, pl.ds(k * 16, 16)][...] = (
                        plsc.load_gather(rows_v.at[g], [iota16 + k * 16, ddv])
                    )

        def write_out(u, tb):
            slot, j = u // lpw, u % lpw
            for si in range(nd):
                pltpu.async_copy(
                    tiles_v.at[tb, si], out_hbm.at[slot, si, lbase + j],
                    osems[tb],
                )

        def wait_out(tb):
            for si in range(nd):
                pltpu.make_async_copy(
                    tiles_v.at[tb, si], out_hbm.at[0, si, 0], osems[tb]
                ).wait()

        for g in range(_G):
            issue_gather(g, g)
        for g in range(_G):
            drain_gather(g)
            if g >= 2:
                wait_out(g % 2)
            transpose(g, g % 2)
            issue_gather(g + _G, g)
            write_out(g, g % 2)

        @pl.loop(_G, nunits - _G, step=_G)
        def _(u):
            for g in range(_G):
                drain_gather(g)
                wait_out(g % 2)
                transpose(g, g % 2)
                issue_gather(u + g + _G, g)
                write_out(u + g, g % 2)

        for g in range(_G):
            drain_gather(g)
            wait_out(g % 2)
            transpose(g, g % 2)
            write_out(nunits - _G + g, g % 2)
        wait_out(0)
        wait_out(1)

    kout = _gather(table, idx_t)
    return kout.transpose(2, 4, 0, 1, 3).reshape(b, s, d)
